# baseline (speedup 1.0000x reference)
import functools

import jax
import jax.numpy as jnp
from jax import lax
from jax.experimental import pallas as pl
from jax.experimental.pallas import tpu as pltpu
from jax.experimental.pallas import tpu_sc as plsc

_BATCH = 16384
_D = 32
_L = 16
_NC = 2
_NS = 16
_NW = _NC * _NS
_BPW = _BATCH // _NW
_CHUNK = 256
_PACK = 128 // _D


def _body(user_hbm, item_hbm, uf_hbm, if_hbm, out_hbm,
          uidx_v, iidx_v, uslice_v, islice_v, urows_v, irows_v, out_v,
          sem_u, sem_i):
    wid = lax.axis_index("s") * _NC + lax.axis_index("c")
    base = wid * _BPW

    pltpu.sync_copy(user_hbm.at[pl.ds(base, _BPW)], uidx_v)
    pltpu.sync_copy(item_hbm.at[pl.ds(base, _BPW)], iidx_v)

    def slice_ids(j, carry):
        s = pl.ds(j * _L, _L)
        uslice_v[s] = lax.shift_right_logical(uidx_v[s], 2)
        islice_v[s] = lax.shift_right_logical(iidx_v[s], 2)
        return carry
    lax.fori_loop(0, _BPW // _L, slice_ids, 0)

    lane = lax.iota(jnp.int32, _L)

    for c in range(_BPW // _CHUNK):
        cbase = c * _CHUNK
        cu = pltpu.async_copy(
            uf_hbm.at[uslice_v.at[pl.ds(cbase, _CHUNK)]], urows_v, sem_u)
        ci = pltpu.async_copy(
            if_hbm.at[islice_v.at[pl.ds(cbase, _CHUNK)]], irows_v, sem_i)
        cu.wait()
        ci.wait()

        def group(g, carry):
            rows = g * _L + lane
            uoff = (uidx_v[pl.ds(cbase + g * _L, _L)] & (_PACK - 1)) * _D
            ioff = (iidx_v[pl.ds(cbase + g * _L, _L)] & (_PACK - 1)) * _D
            acc = jnp.zeros((_L,), jnp.float32)
            for d in range(_D):
                u = plsc.load_gather(urows_v, [rows, uoff + d])
                i = plsc.load_gather(irows_v, [rows, ioff + d])
                acc = acc + u * i
            out_v[pl.ds(cbase + g * _L, _L)] = acc
            return carry

        lax.fori_loop(0, _CHUNK // _L, group, 0)

    pltpu.sync_copy(out_v, out_hbm.at[pl.ds(base, _BPW)])


@jax.jit
def kernel(user, item, user_factors, item_factors):
    n_users, d = user_factors.shape
    n_items, _ = item_factors.shape
    uf_wide = user_factors.reshape(n_users // _PACK, _PACK * d)
    if_wide = item_factors.reshape(n_items // _PACK, _PACK * d)
    run = functools.partial(
        pl.kernel,
        out_type=jax.ShapeDtypeStruct((_BATCH,), jnp.float32),
        mesh=plsc.VectorSubcoreMesh(
            core_axis_name="c", subcore_axis_name="s",
            num_cores=_NC, num_subcores=_NS),
        scratch_types=[
            pltpu.VMEM((_BPW,), jnp.int32),
            pltpu.VMEM((_BPW,), jnp.int32),
            pltpu.VMEM((_BPW,), jnp.int32),
            pltpu.VMEM((_BPW,), jnp.int32),
            pltpu.VMEM((_CHUNK, _PACK * _D), jnp.float32),
            pltpu.VMEM((_CHUNK, _PACK * _D), jnp.float32),
            pltpu.VMEM((_BPW,), jnp.float32),
            pltpu.SemaphoreType.DMA,
            pltpu.SemaphoreType.DMA,
        ],
        compiler_params=pltpu.CompilerParams(needs_layout_passes=False),
    )(_body)
    return run(user, item, uf_wide, if_wide)


# speedup vs baseline: 1.0036x; 1.0036x over previous
import functools

import jax
import jax.numpy as jnp
from jax import lax
from jax.experimental import pallas as pl
from jax.experimental.pallas import tpu as pltpu
from jax.experimental.pallas import tpu_sc as plsc

_BATCH = 16384
_D = 32
_L = 16
_NC = 2
_NS = 16
_NW = _NC * _NS
_BPW = _BATCH // _NW
_CHUNK = 256
_PACK = 128 // _D


def _body(user_hbm, item_hbm, uf_hbm, if_hbm, out_hbm,
          uidx_v, iidx_v, uslice_v, islice_v, urows_v, irows_v, out_v,
          sem_u, sem_i):
    wid = lax.axis_index("s") * _NC + lax.axis_index("c")
    base = wid * _BPW

    pltpu.sync_copy(user_hbm.at[pl.ds(base, _BPW)], uidx_v)
    pltpu.sync_copy(item_hbm.at[pl.ds(base, _BPW)], iidx_v)

    def slice_ids(j, carry):
        s = pl.ds(j * _L, _L)
        uslice_v[s] = lax.shift_right_logical(uidx_v[s], 2)
        islice_v[s] = lax.shift_right_logical(iidx_v[s], 2)
        return carry
    lax.fori_loop(0, _BPW // _L, slice_ids, 0)

    lane = lax.iota(jnp.int32, _L)

    for c in range(_BPW // _CHUNK):
        cbase = c * _CHUNK
        cu = pltpu.async_copy(
            uf_hbm.at[uslice_v.at[pl.ds(cbase, _CHUNK)]], urows_v, sem_u)
        ci = pltpu.async_copy(
            if_hbm.at[islice_v.at[pl.ds(cbase, _CHUNK)]], irows_v, sem_i)
        cu.wait()
        ci.wait()

        def group(g, carry):
            rows = g * _L + lane
            uoff = (uidx_v[pl.ds(cbase + g * _L, _L)] & (_PACK - 1)) * _D
            ioff = (iidx_v[pl.ds(cbase + g * _L, _L)] & (_PACK - 1)) * _D
            acc = jnp.zeros((_L,), jnp.float32)
            for d in range(_D):
                u = plsc.load_gather(urows_v, [rows, uoff + d])
                i = plsc.load_gather(irows_v, [rows, ioff + d])
                acc = acc + u * i
            out_v[pl.ds(cbase + g * _L, _L)] = acc
            return carry

        lax.fori_loop(0, _CHUNK // _L, group, 0)

    pltpu.sync_copy(out_v, out_hbm.at[pl.ds(base, _BPW)])


@jax.jit
def kernel(user, item, user_factors, item_factors):
    n_users, d = user_factors.shape
    n_items, _ = item_factors.shape
    uf_wide = user_factors.reshape(n_users // _PACK, _PACK * d)
    if_wide = item_factors.reshape(n_items // _PACK, _PACK * d)
    run = functools.partial(
        pl.kernel,
        out_type=jax.ShapeDtypeStruct((_BATCH,), jnp.float32),
        mesh=plsc.VectorSubcoreMesh(
            core_axis_name="c", subcore_axis_name="s",
            num_cores=_NC, num_subcores=_NS),
        scratch_types=[
            pltpu.VMEM((_BPW,), jnp.int32),
            pltpu.VMEM((_BPW,), jnp.int32),
            pltpu.VMEM((_BPW,), jnp.int32),
            pltpu.VMEM((_BPW,), jnp.int32),
            pltpu.VMEM((_CHUNK, _PACK * _D), jnp.float32),
            pltpu.VMEM((_CHUNK, _PACK * _D), jnp.float32),
            pltpu.VMEM((_BPW,), jnp.float32),
            pltpu.SemaphoreType.DMA,
            pltpu.SemaphoreType.DMA,
        ],
        compiler_params=pltpu.CompilerParams(
            needs_layout_passes=False, use_tc_tiling_on_sc=True),
    )(_body)
    return run(user, item, uf_wide, if_wide)


# speedup vs baseline: 2.4066x; 2.3979x over previous
import functools

import jax
import jax.numpy as jnp
from jax import lax
from jax.experimental import pallas as pl
from jax.experimental.pallas import tpu as pltpu
from jax.experimental.pallas import tpu_sc as plsc

_BATCH = 16384
_D = 32
_L = 16
_NC = 2
_NS = 16
_NW = _NC * _NS
_BPW = _BATCH // _NW
_CHUNK = 64
_PACK = 128 // _D
_SLOTS = 8


def _body(user_hbm, item_hbm, uft_hbm, ifw_hbm, out_hbm,
          uidx_v, iidx_v, islice_v, ublk_v, urows_v, irows_v,
          out_v, usem, sem_i):
    wid = lax.axis_index("s") * _NC + lax.axis_index("c")
    base = wid * _BPW

    pltpu.sync_copy(user_hbm.at[pl.ds(base, _BPW)], uidx_v)
    pltpu.sync_copy(item_hbm.at[pl.ds(base, _BPW)], iidx_v)

    def slice_ids(j, carry):
        s = pl.ds(j * _L, _L)
        islice_v[s] = lax.shift_right_logical(iidx_v[s], 2)
        return carry
    lax.fori_loop(0, _BPW // _L, slice_ids, 0)

    lane = lax.iota(jnp.int32, _L)

    def group_u(g, carry):
        uv = uidx_v[pl.ds(g * _L, _L)]
        rs = [uv[k] for k in range(_L)]
        for h in range(_L // _SLOTS):
            for k in range(_SLOTS):
                r = rs[h * _SLOTS + k]
                blk = pl.multiple_of((r >> 7) * 128, 128)
                pltpu.async_copy(
                    uft_hbm.at[:, pl.ds(blk, 128)], ublk_v.at[k], usem.at[k])
            for k in range(_SLOTS):
                pltpu.make_async_copy(
                    uft_hbm.at[:, pl.ds(0, 128)], ublk_v.at[k], usem.at[k]
                ).wait()
                r = rs[h * _SLOTS + k]
                col = jnp.full((_L,), r & 127, jnp.int32)
                u0 = plsc.load_gather(ublk_v.at[k], [lane, col])
                u1 = plsc.load_gather(ublk_v.at[k], [lane + _L, col])
                b = g * _L + h * _SLOTS + k
                urows_v[b, pl.ds(0, _L)] = u0
                urows_v[b, pl.ds(_L, _L)] = u1
        return carry
    lax.fori_loop(0, _BPW // _L, group_u, 0)

    for c in range(_BPW // _CHUNK):
        cbase = c * _CHUNK
        ci = pltpu.async_copy(
            ifw_hbm.at[islice_v.at[pl.ds(cbase, _CHUNK)]], irows_v, sem_i)
        ci.wait()

        def group(g, carry):
            rows = g * _L + lane
            ioff = (iidx_v[pl.ds(cbase + g * _L, _L)] & (_PACK - 1)) * _D
            urow = cbase + rows
            acc = jnp.zeros((_L,), jnp.float32)
            for d in range(_D):
                u = plsc.load_gather(urows_v, [urow, jnp.full((_L,), d, jnp.int32)])
                i = plsc.load_gather(irows_v, [rows, ioff + d])
                acc = acc + u * i
            out_v[pl.ds(cbase + g * _L, _L)] = acc
            return carry

        lax.fori_loop(0, _CHUNK // _L, group, 0)

    pltpu.sync_copy(out_v, out_hbm.at[pl.ds(base, _BPW)])


@jax.jit
def kernel(user, item, user_factors, item_factors):
    n_items, d = item_factors.shape
    uf_t = user_factors.T
    if_wide = item_factors.reshape(n_items // _PACK, _PACK * d)
    run = functools.partial(
        pl.kernel,
        out_type=jax.ShapeDtypeStruct((_BATCH,), jnp.float32),
        mesh=plsc.VectorSubcoreMesh(
            core_axis_name="c", subcore_axis_name="s",
            num_cores=_NC, num_subcores=_NS),
        scratch_types=[
            pltpu.VMEM((_BPW,), jnp.int32),
            pltpu.VMEM((_BPW,), jnp.int32),
            pltpu.VMEM((_BPW,), jnp.int32),
            pltpu.VMEM((_SLOTS, _D, 128), jnp.float32),
            pltpu.VMEM((_BPW, _D), jnp.float32),
            pltpu.VMEM((_CHUNK, _PACK * _D), jnp.float32),
            pltpu.VMEM((_BPW,), jnp.float32),
            pltpu.SemaphoreType.DMA((_SLOTS,)),
            pltpu.SemaphoreType.DMA,
        ],
        compiler_params=pltpu.CompilerParams(
            needs_layout_passes=False, use_tc_tiling_on_sc=True),
    )(_body)
    return run(user, item, uf_t, if_wide)


# speedup vs baseline: 2.9254x; 1.2156x over previous
import functools

import jax
import jax.numpy as jnp
from jax import lax
from jax.experimental import pallas as pl
from jax.experimental.pallas import tpu as pltpu
from jax.experimental.pallas import tpu_sc as plsc

_BATCH = 16384
_D = 32
_L = 16
_NC = 2
_NS = 16
_NW = _NC * _NS
_BPW = _BATCH // _NW
_CHUNK = 64
_PACK = 128 // _D
_SLOTS = 8


def _body(user_hbm, item_hbm, uft_hbm, ifw_hbm, out_hbm,
          uidx_v, iidx_v, islice_v, ublk_v, urows_v, irows_v,
          out_v, usem, sem_i):
    wid = lax.axis_index("s") * _NC + lax.axis_index("c")
    base = wid * _BPW

    pltpu.sync_copy(user_hbm.at[pl.ds(base, _BPW)], uidx_v)
    pltpu.sync_copy(item_hbm.at[pl.ds(base, _BPW)], iidx_v)

    def slice_ids(j, carry):
        s = pl.ds(j * _L, _L)
        islice_v[s] = lax.shift_right_logical(iidx_v[s], 2)
        return carry
    lax.fori_loop(0, _BPW // _L, slice_ids, 0)

    lane = lax.iota(jnp.int32, _L)

    def fire_i(c, buf):
        return pltpu.async_copy(
            ifw_hbm.at[islice_v.at[pl.ds(c * _CHUNK, _CHUNK)]],
            irows_v.at[buf], sem_i.at[buf])

    fire_i(0, 0)

    def extract8(o):
        g16 = jnp.minimum((o // 2) * _L, _BPW - _L)
        uv = uidx_v[pl.ds(g16, _L)]

        def lo():
            return tuple(uv[k] for k in range(_SLOTS))

        def hi():
            return tuple(uv[k + _SLOTS] for k in range(_SLOTS))

        return lax.cond(o % 2 == 0, lo, hi)

    def fire_u(r, slot):
        blk = pl.multiple_of((r >> 7) * 128, 128)
        pltpu.async_copy(
            uft_hbm.at[:, pl.ds(blk, 128)], ublk_v.at[slot], usem.at[slot])

    rs0 = extract8(0)
    for k in range(_SLOTS):
        fire_u(rs0[k], k)

    _NOCT = _BPW // _SLOTS

    def octet(o, rs):
        rs_next = extract8(o + 1)
        for k in range(_SLOTS):
            pltpu.make_async_copy(
                uft_hbm.at[:, pl.ds(0, 128)], ublk_v.at[k], usem.at[k]
            ).wait()
            col = jnp.full((_L,), rs[k] & 127, jnp.int32)
            u0 = plsc.load_gather(ublk_v.at[k], [lane, col])
            u1 = plsc.load_gather(ublk_v.at[k], [lane + _L, col])
            b = o * _SLOTS + k
            urows_v[b, pl.ds(0, _L)] = u0
            urows_v[b, pl.ds(_L, _L)] = u1

            @pl.when(o < _NOCT - 1)
            def _():
                fire_u(rs_next[k], k)
        return rs_next
    lax.fori_loop(0, _NOCT, octet, rs0)

    for c in range(_BPW // _CHUNK):
        buf = c % 2
        cbase = c * _CHUNK
        pltpu.make_async_copy(
            ifw_hbm.at[islice_v.at[pl.ds(0, _CHUNK)]],
            irows_v.at[buf], sem_i.at[buf]).wait()
        if c + 1 < _BPW // _CHUNK:
            fire_i(c + 1, 1 - buf)

        def group(g, carry):
            rows = g * _L + lane
            ioff = (iidx_v[pl.ds(cbase + g * _L, _L)] & (_PACK - 1)) * _D
            urow = cbase + rows
            acc = jnp.zeros((_L,), jnp.float32)
            for d in range(_D):
                u = plsc.load_gather(urows_v, [urow, jnp.full((_L,), d, jnp.int32)])
                i = plsc.load_gather(irows_v.at[buf], [rows, ioff + d])
                acc = acc + u * i
            out_v[pl.ds(cbase + g * _L, _L)] = acc
            return carry

        lax.fori_loop(0, _CHUNK // _L, group, 0)

    pltpu.sync_copy(out_v, out_hbm.at[pl.ds(base, _BPW)])


@jax.jit
def kernel(user, item, user_factors, item_factors):
    n_items, d = item_factors.shape
    uf_t = user_factors.T
    if_wide = item_factors.reshape(n_items // _PACK, _PACK * d)
    run = functools.partial(
        pl.kernel,
        out_type=jax.ShapeDtypeStruct((_BATCH,), jnp.float32),
        mesh=plsc.VectorSubcoreMesh(
            core_axis_name="c", subcore_axis_name="s",
            num_cores=_NC, num_subcores=_NS),
        scratch_types=[
            pltpu.VMEM((_BPW,), jnp.int32),
            pltpu.VMEM((_BPW,), jnp.int32),
            pltpu.VMEM((_BPW,), jnp.int32),
            pltpu.VMEM((_SLOTS, _D, 128), jnp.float32),
            pltpu.VMEM((_BPW, _D), jnp.float32),
            pltpu.VMEM((2, _CHUNK, _PACK * _D), jnp.float32),
            pltpu.VMEM((_BPW,), jnp.float32),
            pltpu.SemaphoreType.DMA((_SLOTS,)),
            pltpu.SemaphoreType.DMA((2,)),
        ],
        compiler_params=pltpu.CompilerParams(
            needs_layout_passes=False, use_tc_tiling_on_sc=True),
    )(_body)
    return run(user, item, uf_t, if_wide)
